# Initial kernel scaffold; baseline (speedup 1.0000x reference)
import jax
import jax.numpy as jnp
from jax.experimental import pallas as pl


def kernel(x, edge_index, edge_weight, W_in, b_in, W_orin, b_orin, W_self, W_neigh, b_sage, W_out, b_out):
    raise NotImplementedError("write your pallas kernel here")


# speedup vs baseline: 11.2455x; 11.2455x over previous
import functools

import jax
import jax.numpy as jnp
from jax import lax
from jax.experimental import pallas as pl
from jax.experimental.pallas import tpu as pltpu
from jax.experimental.pallas import tpu_sc as plsc

N = 10000
E = 320000
U = 128
T_IN = 12
T_OUT = 12

NC = 2
NS = 16
NW = NC * NS
CH = 80
EROWS = E // CH
RPT = EROWS // NW
STAGE = 25
NSTAGES = RPT // STAGE
NPAD = 10240
NPT = NPAD // NS

_GATHER_DNUMS = lax.GatherDimensionNumbers(
    offset_dims=(), collapsed_slice_dims=(0,), start_index_map=(0,))


def _sc_aggregate(x_pad, src_r, dst_r, w_r):
    mesh = plsc.VectorSubcoreMesh(core_axis_name="c", subcore_axis_name="s")

    @functools.partial(
        pl.kernel,
        mesh=mesh,
        compiler_params=pltpu.CompilerParams(use_tc_tiling_on_sc=False),
        out_type=jax.ShapeDtypeStruct((NC, NS, NPT, 16), jnp.float32),
        scratch_types=[
            pltpu.VMEM((STAGE, CH), jnp.int32),
            pltpu.VMEM((STAGE, CH), jnp.int32),
            pltpu.VMEM((STAGE, CH), jnp.float32),
            pltpu.VMEM((CH, 16), jnp.float32),
            pltpu.VMEM((NPT, 16), jnp.float32),
            pltpu.VMEM((16,), jnp.float32),
            pltpu.VMEM_SHARED((NPAD, 16), jnp.float32),
            pltpu.SemaphoreType.DMA,
        ],
    )
    def agg(x_hbm, src_hbm, dst_hbm, w_hbm, m13_hbm, out_hbm,
            src_v, dst_v, w_v, rows_v, zbuf, m13_v, acc_sh, sem):
        c = lax.axis_index("c")
        s = lax.axis_index("s")
        zero16 = jnp.zeros((16,), jnp.float32)
        pltpu.sync_copy(m13_hbm, m13_v)

        def zrow(i, carry):
            zbuf[i] = zero16
            return carry
        lax.fori_loop(0, NPT, zrow, 0)
        pltpu.sync_copy(zbuf, acc_sh.at[pl.ds(s * NPT, NPT)])
        plsc.subcore_barrier()

        g = c * NS + s
        for st in range(NSTAGES):
            pltpu.sync_copy(src_hbm.at[g, st], src_v)
            pltpu.sync_copy(dst_hbm.at[g, st], dst_v)
            pltpu.sync_copy(w_hbm.at[g, st], w_v)

            def chunk(j, carry):
                pltpu.async_copy(x_hbm.at[src_v.at[j]], rows_v, sem).wait()
                cm13 = m13_v[...]
                for k in range(CH // 16):
                    w16 = w_v[j, pl.ds(k * 16, 16)]
                    for e in range(16):
                        wv = lax.gather(
                            w16, jnp.full((16, 1), e, jnp.int32),
                            _GATHER_DNUMS, (1,),
                            mode=lax.GatherScatterMode.PROMISE_IN_BOUNDS)
                        ee = k * 16 + e
                        row = rows_v[ee]
                        rows_v[ee] = row * wv + cm13
                pltpu.sync_copy(rows_v, acc_sh.at[dst_v.at[j]], add=True)
                return carry
            lax.fori_loop(0, STAGE, chunk, 0)

        plsc.subcore_barrier()
        pltpu.sync_copy(acc_sh.at[pl.ds(s * NPT, NPT)],
                        out_hbm.at[c, s])

    m13 = jnp.zeros((16,), jnp.float32).at[13].set(1.0)
    return agg(x_pad, src_r, dst_r, w_r, m13)


ROWS_B = 1000


def _tc_body(x_ref, acc_ref, Win_ref, bin_ref, Worin_ref, borin_ref,
             Wself_ref, Wneigh_ref, bsage_ref, Wout_ref, bout_ref, o_ref):
    xb = x_ref[...]
    acc = acc_ref[...]
    Win = Win_ref[...]
    bin_ = bin_ref[...]
    Worin = Worin_ref[...]
    borin = borin_ref[...]

    h = jnp.dot(xb, Win, preferred_element_type=jnp.float32) + bin_
    h_geo = jnp.dot(h, Worin, preferred_element_type=jnp.float32) + borin

    a = acc[0] + acc[1]
    W2 = jnp.dot(Win, Worin, preferred_element_type=jnp.float32)
    b2 = jnp.dot(bin_[None, :], Worin,
                 preferred_element_type=jnp.float32)[0] + borin
    agg = (jnp.dot(a[:, :T_IN], W2, preferred_element_type=jnp.float32)
           + a[:, 12:13] * b2[None, :])
    cnt = a[:, 13:14]
    neigh_mean = agg / jnp.maximum(cnt, 1.0)

    hs = (jnp.dot(h_geo, Wself_ref[...], preferred_element_type=jnp.float32)
          + jnp.dot(neigh_mean, Wneigh_ref[...],
                    preferred_element_type=jnp.float32)
          + bsage_ref[...])
    mu = jnp.mean(hs, axis=1, keepdims=True)
    var = jnp.mean((hs - mu) * (hs - mu), axis=1, keepdims=True)
    hn = (hs - mu) * lax.rsqrt(var + 1e-5)
    ha = jnp.where(hn > 0, hn, 0.01 * hn)
    o_ref[...] = (jnp.dot(ha, Wout_ref[...],
                          preferred_element_type=jnp.float32) + bout_ref[...])


def _tc_dense(x, acc, W_in, b_in, W_orin, b_orin, W_self, W_neigh, b_sage,
              W_out, b_out):
    grid = (N // ROWS_B,)
    full = lambda shape: pl.BlockSpec(shape, lambda i: (0,) * len(shape))
    return pl.pallas_call(
        _tc_body,
        grid=grid,
        in_specs=[
            pl.BlockSpec((ROWS_B, T_IN), lambda i: (i, 0)),
            pl.BlockSpec((NC, ROWS_B, 16), lambda i: (0, i, 0)),
            full((T_IN, U)),
            full((U,)),
            full((U, U)),
            full((U,)),
            full((U, U)),
            full((U, U)),
            full((U,)),
            full((U, T_OUT)),
            full((T_OUT,)),
        ],
        out_specs=pl.BlockSpec((ROWS_B, T_OUT), lambda i: (i, 0)),
        out_shape=jax.ShapeDtypeStruct((N, T_OUT), jnp.float32),
    )(x, acc, W_in, b_in, W_orin, b_orin, W_self, W_neigh, b_sage,
      W_out, b_out)


def kernel(x, edge_index, edge_weight, W_in, b_in, W_orin, b_orin, W_self,
           W_neigh, b_sage, W_out, b_out):
    x_pad = jnp.concatenate(
        [x, jnp.ones((N, 1), jnp.float32), jnp.zeros((N, 3), jnp.float32)],
        axis=1)
    src_r = edge_index[0].reshape(NW, NSTAGES, STAGE, CH)
    dst_r = edge_index[1].reshape(NW, NSTAGES, STAGE, CH)
    w_r = edge_weight.reshape(NW, NSTAGES, STAGE, CH)
    acc = _sc_aggregate(x_pad, src_r, dst_r, w_r)
    acc = acc.reshape(NC, NPAD, 16)[:, :N]
    return _tc_dense(x, acc, W_in, b_in, W_orin, b_orin, W_self, W_neigh,
                     b_sage, W_out, b_out)


# speedup vs baseline: 16.0169x; 1.4243x over previous
import functools

import jax
import jax.numpy as jnp
from jax import lax
from jax.experimental import pallas as pl
from jax.experimental.pallas import tpu as pltpu
from jax.experimental.pallas import tpu_sc as plsc

N = 10000
E = 320000
U = 128
T_IN = 12
T_OUT = 12

NC = 2
NS = 16
NW = NC * NS
CH = 80
EROWS = E // CH
RPT = EROWS // NW
NPAD = 10240
NPT = NPAD // NS

_GATHER_DNUMS = lax.GatherDimensionNumbers(
    offset_dims=(), collapsed_slice_dims=(0,), start_index_map=(0,))


def _sc_aggregate(x_pad, src_r, dst_r, w_r):
    mesh = plsc.VectorSubcoreMesh(core_axis_name="c", subcore_axis_name="s")

    @functools.partial(
        pl.kernel,
        mesh=mesh,
        compiler_params=pltpu.CompilerParams(use_tc_tiling_on_sc=False),
        out_type=jax.ShapeDtypeStruct((NC, NS, NPT, 16), jnp.float32),
        scratch_types=[
            pltpu.VMEM((RPT, CH), jnp.int32),
            pltpu.VMEM((RPT, CH), jnp.int32),
            pltpu.VMEM((RPT, CH), jnp.float32),
            pltpu.VMEM((CH, 16), jnp.float32),
            pltpu.VMEM((CH, 16), jnp.float32),
            pltpu.VMEM((NPT, 16), jnp.float32),
            pltpu.VMEM((16,), jnp.float32),
            pltpu.VMEM_SHARED((NPAD, 16), jnp.float32),
            pltpu.SemaphoreType.DMA,
            pltpu.SemaphoreType.DMA,
        ],
    )
    def agg(x_hbm, src_hbm, dst_hbm, w_hbm, m13_hbm, out_hbm,
            src_v, dst_v, w_v, rows_a, rows_b, zbuf, m13_v, acc_sh,
            sem_a, sem_b):
        c = lax.axis_index("c")
        s = lax.axis_index("s")
        zero16 = jnp.zeros((16,), jnp.float32)
        pltpu.sync_copy(m13_hbm, m13_v)

        def zrow(i, carry):
            zbuf[i] = zero16
            return carry
        lax.fori_loop(0, NPT, zrow, 0)
        pltpu.sync_copy(zbuf, acc_sh.at[pl.ds(s * NPT, NPT)])

        g = c * NS + s
        pltpu.sync_copy(src_hbm.at[g], src_v)
        pltpu.sync_copy(dst_hbm.at[g], dst_v)
        pltpu.sync_copy(w_hbm.at[g], w_v)
        plsc.subcore_barrier()

        def fire(chunk_ix, buf, sem):
            pltpu.async_copy(x_hbm.at[src_v.at[chunk_ix]], buf, sem)

        def wait(buf, sem):
            pltpu.make_async_copy(x_hbm.at[src_v.at[0]], buf, sem).wait()

        def process(buf, chunk_ix):
            cm13 = m13_v[...]
            for k in range(CH // 16):
                w16 = w_v[chunk_ix, pl.ds(k * 16, 16)]
                for e in range(16):
                    wv = lax.gather(
                        w16, jnp.full((16, 1), e, jnp.int32),
                        _GATHER_DNUMS, (1,),
                        mode=lax.GatherScatterMode.PROMISE_IN_BOUNDS)
                    ee = k * 16 + e
                    row = buf[ee]
                    buf[ee] = row * wv + cm13
            pltpu.sync_copy(buf, acc_sh.at[dst_v.at[chunk_ix]], add=True)

        fire(0, rows_a, sem_a)

        def body(i, carry):
            c0 = 2 * i
            fire(c0 + 1, rows_b, sem_b)
            wait(rows_a, sem_a)
            process(rows_a, c0)
            fire(c0 + 2, rows_a, sem_a)
            wait(rows_b, sem_b)
            process(rows_b, c0 + 1)
            return carry
        lax.fori_loop(0, (RPT - 1) // 2, body, 0)
        wait(rows_a, sem_a)
        process(rows_a, RPT - 1)

        plsc.subcore_barrier()
        pltpu.sync_copy(acc_sh.at[pl.ds(s * NPT, NPT)],
                        out_hbm.at[c, s])

    m13 = jnp.zeros((16,), jnp.float32).at[13].set(1.0)
    return agg(x_pad, src_r, dst_r, w_r, m13)


ROWS_B = 1000


def _tc_body(x_ref, acc_ref, Win_ref, bin_ref, Worin_ref, borin_ref,
             Wself_ref, Wneigh_ref, bsage_ref, Wout_ref, bout_ref, o_ref):
    xb = x_ref[...]
    acc = acc_ref[...]
    Win = Win_ref[...]
    bin_ = bin_ref[...]
    Worin = Worin_ref[...]
    borin = borin_ref[...]

    h = jnp.dot(xb, Win, preferred_element_type=jnp.float32) + bin_
    h_geo = jnp.dot(h, Worin, preferred_element_type=jnp.float32) + borin

    a = acc[0] + acc[1]
    W2 = jnp.dot(Win, Worin, preferred_element_type=jnp.float32)
    b2 = jnp.dot(bin_[None, :], Worin,
                 preferred_element_type=jnp.float32)[0] + borin
    agg = (jnp.dot(a[:, :T_IN], W2, preferred_element_type=jnp.float32)
           + a[:, 12:13] * b2[None, :])
    cnt = a[:, 13:14]
    neigh_mean = agg / jnp.maximum(cnt, 1.0)

    hs = (jnp.dot(h_geo, Wself_ref[...], preferred_element_type=jnp.float32)
          + jnp.dot(neigh_mean, Wneigh_ref[...],
                    preferred_element_type=jnp.float32)
          + bsage_ref[...])
    mu = jnp.mean(hs, axis=1, keepdims=True)
    var = jnp.mean((hs - mu) * (hs - mu), axis=1, keepdims=True)
    hn = (hs - mu) * lax.rsqrt(var + 1e-5)
    ha = jnp.where(hn > 0, hn, 0.01 * hn)
    o_ref[...] = (jnp.dot(ha, Wout_ref[...],
                          preferred_element_type=jnp.float32) + bout_ref[...])


def _tc_dense(x, acc, W_in, b_in, W_orin, b_orin, W_self, W_neigh, b_sage,
              W_out, b_out):
    grid = (N // ROWS_B,)
    full = lambda shape: pl.BlockSpec(shape, lambda i: (0,) * len(shape))
    return pl.pallas_call(
        _tc_body,
        grid=grid,
        in_specs=[
            pl.BlockSpec((ROWS_B, T_IN), lambda i: (i, 0)),
            pl.BlockSpec((NC, ROWS_B, 16), lambda i: (0, i, 0)),
            full((T_IN, U)),
            full((U,)),
            full((U, U)),
            full((U,)),
            full((U, U)),
            full((U, U)),
            full((U,)),
            full((U, T_OUT)),
            full((T_OUT,)),
        ],
        out_specs=pl.BlockSpec((ROWS_B, T_OUT), lambda i: (i, 0)),
        out_shape=jax.ShapeDtypeStruct((N, T_OUT), jnp.float32),
    )(x, acc, W_in, b_in, W_orin, b_orin, W_self, W_neigh, b_sage,
      W_out, b_out)


def kernel(x, edge_index, edge_weight, W_in, b_in, W_orin, b_orin, W_self,
           W_neigh, b_sage, W_out, b_out):
    x_pad = jnp.concatenate(
        [x, jnp.ones((N, 1), jnp.float32), jnp.zeros((N, 3), jnp.float32)],
        axis=1)
    src_r = edge_index[0].reshape(NW, RPT, CH)
    dst_r = edge_index[1].reshape(NW, RPT, CH)
    w_r = edge_weight.reshape(NW, RPT, CH)
    acc = _sc_aggregate(x_pad, src_r, dst_r, w_r)
    acc = acc.reshape(NC, NPAD, 16)[:, :N]
    return _tc_dense(x, acc, W_in, b_in, W_orin, b_orin, W_self, W_neigh,
                     b_sage, W_out, b_out)


# speedup vs baseline: 16.8563x; 1.0524x over previous
import functools

import jax
import jax.numpy as jnp
from jax import lax
from jax.experimental import pallas as pl
from jax.experimental.pallas import tpu as pltpu
from jax.experimental.pallas import tpu_sc as plsc

N = 10000
E = 320000
U = 128
T_IN = 12
T_OUT = 12

NC = 2
NS = 16
NW = NC * NS
CH = 80
EROWS = E // CH
RPT = EROWS // NW
NPAD = 10240
NPT = NPAD // NS

_GATHER_DNUMS = lax.GatherDimensionNumbers(
    offset_dims=(), collapsed_slice_dims=(0,), start_index_map=(0,))


def _sc_aggregate(x_pad, src_r, dst_r, w_r):
    mesh = plsc.VectorSubcoreMesh(core_axis_name="c", subcore_axis_name="s")

    @functools.partial(
        pl.kernel,
        mesh=mesh,
        compiler_params=pltpu.CompilerParams(use_tc_tiling_on_sc=False),
        out_type=jax.ShapeDtypeStruct((NC, NS, NPT, 16), jnp.float32),
        scratch_types=[
            pltpu.VMEM((RPT, CH), jnp.int32),
            pltpu.VMEM((RPT, CH), jnp.int32),
            pltpu.VMEM((RPT, CH), jnp.float32),
            pltpu.VMEM((CH, 16), jnp.float32),
            pltpu.VMEM((CH, 16), jnp.float32),
            pltpu.VMEM((CH, 16), jnp.float32),
            pltpu.VMEM((CH, 16), jnp.float32),
            pltpu.VMEM((NPT, 16), jnp.float32),
            pltpu.VMEM((16,), jnp.float32),
            pltpu.VMEM_SHARED((NPAD, 16), jnp.float32),
            pltpu.SemaphoreType.DMA,
            pltpu.SemaphoreType.DMA,
            pltpu.SemaphoreType.DMA,
            pltpu.SemaphoreType.DMA,
        ],
    )
    def agg(x_hbm, src_hbm, dst_hbm, w_hbm, m13_hbm, out_hbm,
            src_v, dst_v, w_v, ga, gb, sa, sb, zbuf, m13_v, acc_sh,
            sem_ga, sem_gb, sem_sa, sem_sb):
        c = lax.axis_index("c")
        s = lax.axis_index("s")
        zero16 = jnp.zeros((16,), jnp.float32)
        pltpu.sync_copy(m13_hbm, m13_v)

        def zrow(i, carry):
            zbuf[i] = zero16
            return carry
        lax.fori_loop(0, NPT, zrow, 0)
        pltpu.sync_copy(zbuf, acc_sh.at[pl.ds(s * NPT, NPT)])

        g = c * NS + s
        pltpu.sync_copy(src_hbm.at[g], src_v)
        pltpu.sync_copy(dst_hbm.at[g], dst_v)
        pltpu.sync_copy(w_hbm.at[g], w_v)
        def zsc(i, carry):
            sa[i] = zero16
            sb[i] = zero16
            return carry
        lax.fori_loop(0, CH, zsc, 0)
        plsc.subcore_barrier()

        def fire(chunk_ix, buf, sem):
            pltpu.async_copy(x_hbm.at[src_v.at[chunk_ix]], buf, sem)

        def gwait(buf, sem):
            pltpu.make_async_copy(x_hbm.at[src_v.at[0]], buf, sem).wait()

        def scat(buf, chunk_ix, sem):
            pltpu.async_copy(buf, acc_sh.at[dst_v.at[chunk_ix]], sem,
                             add=True)

        def swait(buf, sem):
            pltpu.make_async_copy(buf, acc_sh.at[dst_v.at[0]], sem).wait()

        def process(gbuf, sbuf, chunk_ix):
            cm13 = m13_v[...]
            for k in range(CH // 16):
                w16 = w_v[chunk_ix, pl.ds(k * 16, 16)]
                for e in range(16):
                    wv = lax.gather(
                        w16, jnp.full((16, 1), e, jnp.int32),
                        _GATHER_DNUMS, (1,),
                        mode=lax.GatherScatterMode.PROMISE_IN_BOUNDS)
                    ee = k * 16 + e
                    sbuf[ee] = gbuf[ee] * wv + cm13

        scat(sa, 0, sem_sa)
        scat(sb, 0, sem_sb)
        fire(0, ga, sem_ga)

        def body(i, carry):
            c0 = 2 * i
            fire(c0 + 1, gb, sem_gb)
            gwait(ga, sem_ga)
            swait(sa, sem_sa)
            process(ga, sa, c0)
            scat(sa, c0, sem_sa)
            fire(c0 + 2, ga, sem_ga)
            gwait(gb, sem_gb)
            swait(sb, sem_sb)
            process(gb, sb, c0 + 1)
            scat(sb, c0 + 1, sem_sb)
            return carry
        lax.fori_loop(0, (RPT - 1) // 2, body, 0)
        gwait(ga, sem_ga)
        swait(sa, sem_sa)
        process(ga, sa, RPT - 1)
        scat(sa, RPT - 1, sem_sa)
        swait(sa, sem_sa)
        swait(sb, sem_sb)

        plsc.subcore_barrier()
        pltpu.sync_copy(acc_sh.at[pl.ds(s * NPT, NPT)],
                        out_hbm.at[c, s])

    m13 = jnp.zeros((16,), jnp.float32).at[13].set(1.0)
    return agg(x_pad, src_r, dst_r, w_r, m13)


ROWS_B = NPT


def _tc_body(x_ref, acc_ref, Win_ref, bin_ref, Worin_ref, borin_ref,
             Wself_ref, Wneigh_ref, bsage_ref, Wout_ref, bout_ref, o_ref):
    xb16 = x_ref[...]
    acc = acc_ref[...]
    Win = Win_ref[...]
    bin_ = bin_ref[...]
    Worin = Worin_ref[...]
    borin = borin_ref[...]

    W2 = jnp.dot(Win, Worin, preferred_element_type=jnp.float32)
    b2 = jnp.dot(bin_[None, :], Worin,
                 preferred_element_type=jnp.float32)[0] + borin
    A_ = jnp.dot(W2, Wself_ref[...], preferred_element_type=jnp.float32)
    B_ = jnp.dot(W2, Wneigh_ref[...], preferred_element_type=jnp.float32)
    bB = jnp.dot(b2[None, :], Wneigh_ref[...],
                 preferred_element_type=jnp.float32)[0]
    bC = jnp.dot(b2[None, :], Wself_ref[...],
                 preferred_element_type=jnp.float32)[0] + bsage_ref[...]

    a = acc[0, 0] + acc[1, 0]
    xb = xb16[:, :T_IN]
    a12 = a[:, :T_IN]
    wsum = a[:, 12:13]
    cnt = a[:, 13:14]
    cntc = jnp.maximum(cnt, 1.0)

    hs = (jnp.dot(xb, A_, preferred_element_type=jnp.float32) + bC
          + (jnp.dot(a12, B_, preferred_element_type=jnp.float32)
             + wsum * bB[None, :]) / cntc)
    mu = jnp.mean(hs, axis=1, keepdims=True)
    var = jnp.mean((hs - mu) * (hs - mu), axis=1, keepdims=True)
    hn = (hs - mu) * lax.rsqrt(var + 1e-5)
    ha = jnp.where(hn > 0, hn, 0.01 * hn)
    o_ref[...] = (jnp.dot(ha, Wout_ref[...],
                          preferred_element_type=jnp.float32) + bout_ref[...])


def _tc_dense(x_pad, acc, W_in, b_in, W_orin, b_orin, W_self, W_neigh,
              b_sage, W_out, b_out):
    grid = (NPAD // ROWS_B,)
    full = lambda shape: pl.BlockSpec(shape, lambda i: (0,) * len(shape))
    return pl.pallas_call(
        _tc_body,
        grid=grid,
        in_specs=[
            pl.BlockSpec((ROWS_B, 16), lambda i: (i, 0)),
            pl.BlockSpec((NC, 1, ROWS_B, 16), lambda i: (0, i, 0, 0)),
            full((T_IN, U)),
            full((U,)),
            full((U, U)),
            full((U,)),
            full((U, U)),
            full((U, U)),
            full((U,)),
            full((U, T_OUT)),
            full((T_OUT,)),
        ],
        out_specs=pl.BlockSpec((ROWS_B, T_OUT), lambda i: (i, 0)),
        out_shape=jax.ShapeDtypeStruct((NPAD, T_OUT), jnp.float32),
    )(x_pad, acc, W_in, b_in, W_orin, b_orin, W_self, W_neigh, b_sage,
      W_out, b_out)


def kernel(x, edge_index, edge_weight, W_in, b_in, W_orin, b_orin, W_self,
           W_neigh, b_sage, W_out, b_out):
    x_pad = jnp.concatenate(
        [x, jnp.ones((N, 1), jnp.float32), jnp.zeros((N, 3), jnp.float32)],
        axis=1)
    x_pad = jnp.pad(x_pad, ((0, NPAD - N), (0, 0)))
    src_r = edge_index[0].reshape(NW, RPT, CH)
    dst_r = edge_index[1].reshape(NW, RPT, CH)
    w_r = edge_weight.reshape(NW, RPT, CH)
    acc = _sc_aggregate(x_pad, src_r, dst_r, w_r)
    out = _tc_dense(x_pad, acc, W_in, b_in, W_orin, b_orin, W_self, W_neigh,
                    b_sage, W_out, b_out)
    return out[:N]


# speedup vs baseline: 24.5339x; 1.4555x over previous
import functools

import jax
import jax.numpy as jnp
from jax import lax
from jax.experimental import pallas as pl
from jax.experimental.pallas import tpu as pltpu
from jax.experimental.pallas import tpu_sc as plsc

N = 10000
E = 320000
U = 128
T_IN = 12
T_OUT = 12

NC = 2
NS = 16
NW = NC * NS
CH = 80
EROWS = E // CH
RPT = EROWS // NW
NPAD = 10240
NPT = NPAD // NS
NBUF = 4

_GATHER_DNUMS = lax.GatherDimensionNumbers(
    offset_dims=(), collapsed_slice_dims=(0,), start_index_map=(0,))


def _sc_aggregate(x_pad, edge_r, w_r):
    mesh = plsc.VectorSubcoreMesh(core_axis_name="c", subcore_axis_name="s")

    @functools.partial(
        pl.kernel,
        mesh=mesh,
        compiler_params=pltpu.CompilerParams(use_tc_tiling_on_sc=False),
        out_type=jax.ShapeDtypeStruct((NC, NS, NPT, 16), jnp.float32),
        scratch_types=[
            pltpu.VMEM((RPT, CH), jnp.int32),
            pltpu.VMEM((RPT, CH), jnp.int32),
            pltpu.VMEM((RPT, CH), jnp.float32),
            [pltpu.VMEM((CH, 16), jnp.float32)] * NBUF,
            [pltpu.VMEM((CH, 16), jnp.float32)] * NBUF,
            pltpu.VMEM((NPT, 16), jnp.float32),
            pltpu.VMEM((16,), jnp.float32),
            pltpu.VMEM_SHARED((NPAD, 16), jnp.float32),
            [pltpu.SemaphoreType.DMA] * NBUF,
            [pltpu.SemaphoreType.DMA] * NBUF,
        ],
    )
    def agg(x_hbm, edge_hbm, w_hbm, m13_hbm, out_hbm,
            src_v, dst_v, w_v, gbufs, sbufs, zbuf, m13_v, acc_sh,
            gsems, ssems):
        c = lax.axis_index("c")
        s = lax.axis_index("s")
        zero16 = jnp.zeros((16,), jnp.float32)
        pltpu.sync_copy(m13_hbm, m13_v)

        def zrow(i, carry):
            zbuf[i] = zero16
            return carry
        lax.fori_loop(0, NPT, zrow, 0)
        pltpu.sync_copy(zbuf, acc_sh.at[pl.ds(s * NPT, NPT)])

        g = c * NS + s
        pltpu.sync_copy(edge_hbm.at[0, g], src_v)
        pltpu.sync_copy(edge_hbm.at[1, g], dst_v)
        pltpu.sync_copy(w_hbm.at[g], w_v)
        def zsc(i, carry):
            for b in range(NBUF):
                sbufs[b][i] = zero16
            return carry
        lax.fori_loop(0, CH, zsc, 0)
        plsc.subcore_barrier()

        def fire(chunk_ix, b):
            if isinstance(chunk_ix, int):
                pltpu.async_copy(x_hbm.at[src_v.at[chunk_ix]], gbufs[b],
                                 gsems[b])
            else:
                @pl.when(chunk_ix < RPT)
                def _():
                    pltpu.async_copy(x_hbm.at[src_v.at[chunk_ix]], gbufs[b],
                                     gsems[b])

        def gwait(b):
            pltpu.make_async_copy(x_hbm.at[src_v.at[0]], gbufs[b],
                                  gsems[b]).wait()

        def scat(b, chunk_ix):
            pltpu.async_copy(sbufs[b], acc_sh.at[dst_v.at[chunk_ix]],
                             ssems[b], add=True)

        def swait(b):
            pltpu.make_async_copy(sbufs[b], acc_sh.at[dst_v.at[0]],
                                  ssems[b]).wait()

        def process(b, chunk_ix):
            cm13 = m13_v[...]
            gbuf = gbufs[b]
            sbuf = sbufs[b]
            for k in range(CH // 16):
                w16 = w_v[chunk_ix, pl.ds(k * 16, 16)]
                for e in range(16):
                    wv = lax.gather(
                        w16, jnp.full((16, 1), e, jnp.int32),
                        _GATHER_DNUMS, (1,),
                        mode=lax.GatherScatterMode.PROMISE_IN_BOUNDS)
                    ee = k * 16 + e
                    sbuf[ee] = gbuf[ee] * wv + cm13

        for b in range(NBUF):
            scat(b, 0)
        for b in range(NBUF - 1):
            fire(b, b)

        def body(i, carry):
            c0 = NBUF * i
            fire(c0 + NBUF - 1, NBUF - 1)
            for b in range(NBUF):
                gwait(b)
                swait(b)
                process(b, c0 + b)
                scat(b, c0 + b)
                if b < NBUF - 1:
                    fire(c0 + NBUF + b, b)
            return carry
        lax.fori_loop(0, RPT // NBUF, body, 0)
        gwait(0)
        swait(0)
        process(0, RPT - 1)
        scat(0, RPT - 1)
        for b in range(NBUF):
            swait(b)

        plsc.subcore_barrier()
        pltpu.sync_copy(acc_sh.at[pl.ds(s * NPT, NPT)],
                        out_hbm.at[c, s])

    m13 = jnp.zeros((16,), jnp.float32).at[13].set(1.0)
    return agg(x_pad, edge_r, w_r, m13)


TCB = 4
ROWS_B = TCB * NPT


def _tc_body(x_ref, acc_ref, Win_ref, bin_ref, Worin_ref, borin_ref,
             Wself_ref, Wneigh_ref, bsage_ref, Wout_ref, bout_ref, o_ref,
             A_s, B_s):
    i = pl.program_id(0)

    @pl.when(i == 0)
    def _fold_weights():
        Win = Win_ref[...]
        Worin = Worin_ref[...]
        W2 = jnp.dot(Win, Worin, preferred_element_type=jnp.float32)
        b2 = jnp.dot(bin_ref[...][None, :], Worin,
                     preferred_element_type=jnp.float32)[0] + borin_ref[...]
        A_ = jnp.dot(W2, Wself_ref[...], preferred_element_type=jnp.float32)
        B_ = jnp.dot(W2, Wneigh_ref[...], preferred_element_type=jnp.float32)
        bC = (jnp.dot(b2[None, :], Wself_ref[...],
                      preferred_element_type=jnp.float32)[0]
              + bsage_ref[...])
        bB = jnp.dot(b2[None, :], Wneigh_ref[...],
                     preferred_element_type=jnp.float32)[0]
        zf = jnp.zeros((3, U), jnp.float32)
        A_s[...] = jnp.concatenate([A_, bC[None, :], zf], axis=0)
        B_s[...] = jnp.concatenate([B_, bB[None, :], zf], axis=0)

    xb16 = x_ref[...]
    acc = acc_ref[...]
    a16 = (acc[0] + acc[1]).reshape(ROWS_B, 16)
    A16 = A_s[...]
    B16 = B_s[...]

    cnt = a16[:, 13:14]
    cntc = jnp.maximum(cnt, 1.0)
    hs = (jnp.dot(xb16, A16, preferred_element_type=jnp.float32)
          + jnp.dot(a16, B16, preferred_element_type=jnp.float32) / cntc)
    mu = jnp.mean(hs, axis=1, keepdims=True)
    var = jnp.mean((hs - mu) * (hs - mu), axis=1, keepdims=True)
    hn = (hs - mu) * lax.rsqrt(var + 1e-5)
    ha = jnp.where(hn > 0, hn, 0.01 * hn)
    o_ref[...] = (jnp.dot(ha, Wout_ref[...],
                          preferred_element_type=jnp.float32) + bout_ref[...])


def _tc_dense(x_pad, acc, W_in, b_in, W_orin, b_orin, W_self, W_neigh,
              b_sage, W_out, b_out):
    grid = (NPAD // ROWS_B,)
    full = lambda shape: pl.BlockSpec(shape, lambda i: (0,) * len(shape))
    return pl.pallas_call(
        _tc_body,
        grid=grid,
        in_specs=[
            pl.BlockSpec((ROWS_B, 16), lambda i: (i, 0)),
            pl.BlockSpec((NC, TCB, NPT, 16), lambda i: (0, i, 0, 0)),
            full((T_IN, U)),
            full((U,)),
            full((U, U)),
            full((U,)),
            full((U, U)),
            full((U, U)),
            full((U,)),
            full((U, T_OUT)),
            full((T_OUT,)),
        ],
        out_specs=pl.BlockSpec((ROWS_B, T_OUT), lambda i: (i, 0)),
        out_shape=jax.ShapeDtypeStruct((NPAD, T_OUT), jnp.float32),
        scratch_shapes=[
            pltpu.VMEM((16, U), jnp.float32),
            pltpu.VMEM((16, U), jnp.float32),
        ],
    )(x_pad, acc, W_in, b_in, W_orin, b_orin, W_self, W_neigh, b_sage,
      W_out, b_out)


def kernel(x, edge_index, edge_weight, W_in, b_in, W_orin, b_orin, W_self,
           W_neigh, b_sage, W_out, b_out):
    x_pad = jnp.concatenate(
        [x, jnp.ones((N, 1), jnp.float32), jnp.zeros((N, 3), jnp.float32)],
        axis=1)
    x_pad = jnp.pad(x_pad, ((0, NPAD - N), (0, 0)))
    edge_r = edge_index.reshape(2, NW, RPT, CH)
    w_r = edge_weight.reshape(NW, RPT, CH)
    acc = _sc_aggregate(x_pad, edge_r, w_r)
    out = _tc_dense(x_pad, acc, W_in, b_in, W_orin, b_orin, W_self, W_neigh,
                    b_sage, W_out, b_out)
    return out[:N]


# speedup vs baseline: 24.8073x; 1.0111x over previous
import functools

import jax
import jax.numpy as jnp
from jax import lax
from jax.experimental import pallas as pl
from jax.experimental.pallas import tpu as pltpu
from jax.experimental.pallas import tpu_sc as plsc

N = 10000
E = 320000
U = 128
T_IN = 12
T_OUT = 12

NC = 2
NS = 16
NW = NC * NS
CH = 80
EROWS = E // CH
RPT = EROWS // NW
NPAD = 10240
NPT = NPAD // NS
NBUF = 4

_GATHER_DNUMS = lax.GatherDimensionNumbers(
    offset_dims=(), collapsed_slice_dims=(0,), start_index_map=(0,))


def _sc_aggregate(x_pad, edge_r, w_r):
    mesh = plsc.VectorSubcoreMesh(core_axis_name="c", subcore_axis_name="s")

    @functools.partial(
        pl.kernel,
        mesh=mesh,
        compiler_params=pltpu.CompilerParams(use_tc_tiling_on_sc=False),
        out_type=jax.ShapeDtypeStruct((NC, NS, NPT, 16), jnp.float32),
        scratch_types=[
            pltpu.VMEM((RPT, CH), jnp.int32),
            pltpu.VMEM((RPT, CH), jnp.int32),
            pltpu.VMEM((RPT, CH), jnp.float32),
            [pltpu.VMEM((CH, 16), jnp.float32)] * NBUF,
            [pltpu.VMEM((CH, 16), jnp.float32)] * NBUF,
            pltpu.VMEM((NPT, 16), jnp.float32),
            pltpu.VMEM((16,), jnp.float32),
            pltpu.VMEM_SHARED((NPAD, 16), jnp.float32),
            [pltpu.SemaphoreType.DMA] * NBUF,
            [pltpu.SemaphoreType.DMA] * NBUF,
        ],
    )
    def agg(x_hbm, edge_hbm, w_hbm, m13_hbm, out_hbm,
            src_v, dst_v, w_v, gbufs, sbufs, zbuf, m13_v, acc_sh,
            gsems, ssems):
        c = lax.axis_index("c")
        s = lax.axis_index("s")
        zero16 = jnp.zeros((16,), jnp.float32)
        pltpu.sync_copy(m13_hbm, m13_v)

        def zrow(i, carry):
            zbuf[i] = zero16
            return carry
        lax.fori_loop(0, NPT, zrow, 0)
        pltpu.sync_copy(zbuf, acc_sh.at[pl.ds(s * NPT, NPT)])

        g = c * NS + s
        pltpu.sync_copy(edge_hbm.at[0, g], src_v)
        pltpu.sync_copy(edge_hbm.at[1, g], dst_v)
        pltpu.sync_copy(w_hbm.at[g], w_v)
        def zsc(i, carry):
            for b in range(NBUF):
                sbufs[b][i] = zero16
            return carry
        lax.fori_loop(0, CH, zsc, 0)
        plsc.subcore_barrier()

        def fire(chunk_ix, b):
            if isinstance(chunk_ix, int):
                pltpu.async_copy(x_hbm.at[src_v.at[chunk_ix]], gbufs[b],
                                 gsems[b])
            else:
                @pl.when(chunk_ix < RPT)
                def _():
                    pltpu.async_copy(x_hbm.at[src_v.at[chunk_ix]], gbufs[b],
                                     gsems[b])

        def gwait(b):
            pltpu.make_async_copy(x_hbm.at[src_v.at[0]], gbufs[b],
                                  gsems[b]).wait()

        def scat(b, chunk_ix):
            pltpu.async_copy(sbufs[b], acc_sh.at[dst_v.at[chunk_ix]],
                             ssems[b], add=True)

        def swait(b):
            pltpu.make_async_copy(sbufs[b], acc_sh.at[dst_v.at[0]],
                                  ssems[b]).wait()

        def process(b, chunk_ix):
            cm13 = m13_v[...]
            gbuf = gbufs[b]
            sbuf = sbufs[b]
            for k in range(CH // 16):
                w16 = w_v[chunk_ix, pl.ds(k * 16, 16)]
                for e in range(16):
                    wv = lax.gather(
                        w16, jnp.full((16, 1), e, jnp.int32),
                        _GATHER_DNUMS, (1,),
                        mode=lax.GatherScatterMode.PROMISE_IN_BOUNDS)
                    ee = k * 16 + e
                    sbuf[ee] = gbuf[ee] * wv + cm13

        for b in range(NBUF):
            scat(b, 0)
        for b in range(NBUF - 1):
            fire(b, b)

        def body(i, carry):
            c0 = NBUF * i
            fire(c0 + NBUF - 1, NBUF - 1)
            for b in range(NBUF):
                gwait(b)
                swait(b)
                process(b, c0 + b)
                scat(b, c0 + b)
                if b < NBUF - 1:
                    fire(c0 + NBUF + b, b)
            return carry
        lax.fori_loop(0, RPT // NBUF, body, 0)
        gwait(0)
        swait(0)
        process(0, RPT - 1)
        scat(0, RPT - 1)
        for b in range(NBUF):
            swait(b)

        plsc.subcore_barrier()
        pltpu.sync_copy(acc_sh.at[pl.ds(s * NPT, NPT)],
                        out_hbm.at[c, s])

    m13 = jnp.zeros((16,), jnp.float32).at[13].set(1.0)
    return agg(x_pad, edge_r, w_r, m13)


TCB = 4
ROWS_B = TCB * NPT


def _tc_body(x_ref, acc_ref, Win_ref, bin_ref, Worin_ref, borin_ref,
             Wself_ref, Wneigh_ref, bsage_ref, Wout_ref, bout_ref, o_ref,
             A_s, B_s):
    i = pl.program_id(0)

    @pl.when(i == 0)
    def _fold_weights():
        Win = Win_ref[...]
        Worin = Worin_ref[...]
        W2 = jnp.dot(Win, Worin, preferred_element_type=jnp.float32)
        b2 = jnp.dot(bin_ref[...][None, :], Worin,
                     preferred_element_type=jnp.float32)[0] + borin_ref[...]
        A_ = jnp.dot(W2, Wself_ref[...], preferred_element_type=jnp.float32)
        B_ = jnp.dot(W2, Wneigh_ref[...], preferred_element_type=jnp.float32)
        bC = (jnp.dot(b2[None, :], Wself_ref[...],
                      preferred_element_type=jnp.float32)[0]
              + bsage_ref[...])
        bB = jnp.dot(b2[None, :], Wneigh_ref[...],
                     preferred_element_type=jnp.float32)[0]
        zf = jnp.zeros((3, U), jnp.float32)
        A_s[...] = jnp.concatenate([A_, bC[None, :], zf], axis=0)
        B_s[...] = jnp.concatenate([B_, bB[None, :], zf], axis=0)

    xb16 = x_ref[...]
    acc = acc_ref[...]
    a16 = (acc[0] + acc[1]).reshape(ROWS_B, 16)
    A16 = A_s[...]
    B16 = B_s[...]

    cnt = a16[:, 13:14]
    cntc = jnp.maximum(cnt, 1.0)
    hs = (jnp.dot(xb16, A16, preferred_element_type=jnp.float32)
          + jnp.dot(a16, B16, preferred_element_type=jnp.float32) / cntc)
    mu = jnp.mean(hs, axis=1, keepdims=True)
    var = jnp.mean((hs - mu) * (hs - mu), axis=1, keepdims=True)
    hn = (hs - mu) * lax.rsqrt(var + 1e-5)
    ha = jnp.where(hn > 0, hn, 0.01 * hn)
    o_ref[...] = (jnp.dot(ha, Wout_ref[...],
                          preferred_element_type=jnp.float32) + bout_ref[...])


def _tc_dense(x_pad, acc, W_in, b_in, W_orin, b_orin, W_self, W_neigh,
              b_sage, W_out, b_out):
    grid = (NPAD // ROWS_B,)
    full = lambda shape: pl.BlockSpec(shape, lambda i: (0,) * len(shape))
    return pl.pallas_call(
        _tc_body,
        grid=grid,
        in_specs=[
            pl.BlockSpec((ROWS_B, 16), lambda i: (i, 0)),
            pl.BlockSpec((NC, TCB, NPT, 16), lambda i: (0, i, 0, 0)),
            full((T_IN, U)),
            full((U,)),
            full((U, U)),
            full((U,)),
            full((U, U)),
            full((U, U)),
            full((U,)),
            full((U, T_OUT)),
            full((T_OUT,)),
        ],
        out_specs=pl.BlockSpec((ROWS_B, T_OUT), lambda i: (i, 0)),
        out_shape=jax.ShapeDtypeStruct((N, T_OUT), jnp.float32),
        scratch_shapes=[
            pltpu.VMEM((16, U), jnp.float32),
            pltpu.VMEM((16, U), jnp.float32),
        ],
    )(x_pad, acc, W_in, b_in, W_orin, b_orin, W_self, W_neigh, b_sage,
      W_out, b_out)


def kernel(x, edge_index, edge_weight, W_in, b_in, W_orin, b_orin, W_self,
           W_neigh, b_sage, W_out, b_out):
    x_pad = jnp.concatenate(
        [x, jnp.ones((N, 1), jnp.float32), jnp.zeros((N, 3), jnp.float32)],
        axis=1)
    x_pad = jnp.pad(x_pad, ((0, NPAD - N), (0, 0)))
    edge_r = edge_index.reshape(2, NW, RPT, CH)
    w_r = edge_weight.reshape(NW, RPT, CH)
    acc = _sc_aggregate(x_pad, edge_r, w_r)
    return _tc_dense(x_pad, acc, W_in, b_in, W_orin, b_orin, W_self, W_neigh,
                     b_sage, W_out, b_out)


# speedup vs baseline: 25.7141x; 1.0366x over previous
import functools

import jax
import jax.numpy as jnp
from jax import lax
from jax.experimental import pallas as pl
from jax.experimental.pallas import tpu as pltpu
from jax.experimental.pallas import tpu_sc as plsc

N = 10000
E = 320000
U = 128
T_IN = 12
T_OUT = 12

NC = 2
NS = 16
NW = NC * NS
CH = 80
EROWS = E // CH
RPT = EROWS // NW
NPAD = 10240
NPT = NPAD // NS
NBUF = 4

_GATHER_DNUMS = lax.GatherDimensionNumbers(
    offset_dims=(), collapsed_slice_dims=(0,), start_index_map=(0,))


def _sc_aggregate(x_pad, edge_r, w_r):
    mesh = plsc.VectorSubcoreMesh(core_axis_name="c", subcore_axis_name="s")

    @functools.partial(
        pl.kernel,
        mesh=mesh,
        compiler_params=pltpu.CompilerParams(use_tc_tiling_on_sc=False),
        out_type=jax.ShapeDtypeStruct((NC, NS, NPT, 16), jnp.float32),
        scratch_types=[
            pltpu.VMEM((RPT, CH), jnp.int32),
            pltpu.VMEM((RPT, CH), jnp.int32),
            pltpu.VMEM((RPT, CH), jnp.float32),
            [pltpu.VMEM((CH, 16), jnp.float32)] * NBUF,
            [pltpu.VMEM((CH, 16), jnp.float32)] * NBUF,
            pltpu.VMEM((NPT, 16), jnp.float32),
            pltpu.VMEM((16,), jnp.float32),
            pltpu.VMEM_SHARED((NPAD, 16), jnp.float32),
            [pltpu.SemaphoreType.DMA] * NBUF,
            [pltpu.SemaphoreType.DMA] * NBUF,
        ],
    )
    def agg(x_hbm, edge_hbm, w_hbm, m13_hbm, out_hbm,
            src_v, dst_v, w_v, gbufs, sbufs, zbuf, m13_v, acc_sh,
            gsems, ssems):
        c = lax.axis_index("c")
        s = lax.axis_index("s")
        zero16 = jnp.zeros((16,), jnp.float32)
        pltpu.sync_copy(m13_hbm, m13_v)

        def zrow(i, carry):
            zbuf[i] = zero16
            return carry
        lax.fori_loop(0, NPT, zrow, 0)
        pltpu.sync_copy(zbuf, acc_sh.at[pl.ds(s * NPT, NPT)])

        g = c * NS + s
        pltpu.sync_copy(edge_hbm.at[0, g], src_v)
        pltpu.sync_copy(edge_hbm.at[1, g], dst_v)
        pltpu.sync_copy(w_hbm.at[g], w_v)
        def zsc(i, carry):
            for b in range(NBUF):
                sbufs[b][i] = zero16
            return carry
        lax.fori_loop(0, CH, zsc, 0)
        plsc.subcore_barrier()

        def fire(chunk_ix, b):
            if isinstance(chunk_ix, int):
                pltpu.async_copy(x_hbm.at[src_v.at[chunk_ix]], gbufs[b],
                                 gsems[b])
            else:
                @pl.when(chunk_ix < RPT)
                def _():
                    pltpu.async_copy(x_hbm.at[src_v.at[chunk_ix]], gbufs[b],
                                     gsems[b])

        def gwait(b):
            pltpu.make_async_copy(x_hbm.at[src_v.at[0]], gbufs[b],
                                  gsems[b]).wait()

        def scat(b, chunk_ix):
            pltpu.async_copy(sbufs[b], acc_sh.at[dst_v.at[chunk_ix]],
                             ssems[b], add=True)

        def swait(b):
            pltpu.make_async_copy(sbufs[b], acc_sh.at[dst_v.at[0]],
                                  ssems[b]).wait()

        def process(b, chunk_ix):
            cm13 = m13_v[...]
            gbuf = gbufs[b]
            sbuf = sbufs[b]
            for k in range(CH // 16):
                w16 = w_v[chunk_ix, pl.ds(k * 16, 16)]
                for e in range(16):
                    wv = lax.gather(
                        w16, jnp.full((16, 1), e, jnp.int32),
                        _GATHER_DNUMS, (1,),
                        mode=lax.GatherScatterMode.PROMISE_IN_BOUNDS)
                    ee = k * 16 + e
                    sbuf[ee] = gbuf[ee] * wv + cm13

        for b in range(NBUF):
            scat(b, 0)
        for b in range(NBUF - 1):
            fire(b, b)

        def body(i, carry):
            c0 = NBUF * i
            fire(c0 + NBUF - 1, NBUF - 1)
            for b in range(NBUF):
                gwait(b)
                swait(b)
                process(b, c0 + b)
                scat(b, c0 + b)
                if b < NBUF - 1:
                    fire(c0 + NBUF + b, b)
            return carry
        lax.fori_loop(0, RPT // NBUF, body, 0)
        gwait(0)
        swait(0)
        process(0, RPT - 1)
        scat(0, RPT - 1)
        for b in range(NBUF):
            swait(b)

        plsc.subcore_barrier()
        pltpu.sync_copy(acc_sh.at[pl.ds(s * NPT, NPT)],
                        out_hbm.at[c, s])

    m13 = jnp.zeros((16,), jnp.float32).at[13].set(1.0)
    return agg(x_pad, edge_r, w_r, m13)


NR8 = NPAD // 8


def _tc_body(x_ref, acc_ref, Win_ref, bin_ref, Worin_ref, borin_ref,
             Wself_ref, Wneigh_ref, bsage_ref, Wout_ref, bout_ref, o_ref):
    Win = Win_ref[...]
    Worin = Worin_ref[...]
    W2 = jnp.dot(Win, Worin, preferred_element_type=jnp.float32)
    b2 = jnp.dot(bin_ref[...][None, :], Worin,
                 preferred_element_type=jnp.float32)[0] + borin_ref[...]
    A_ = jnp.dot(W2, Wself_ref[...], preferred_element_type=jnp.float32)
    B_ = jnp.dot(W2, Wneigh_ref[...], preferred_element_type=jnp.float32)
    bC = (jnp.dot(b2[None, :], Wself_ref[...],
                  preferred_element_type=jnp.float32)[0] + bsage_ref[...])
    bB = jnp.dot(b2[None, :], Wneigh_ref[...],
                 preferred_element_type=jnp.float32)[0]
    zf = jnp.zeros((3, U), jnp.float32)
    A16 = jnp.concatenate([A_, bC[None, :], zf], axis=0)
    B16 = jnp.concatenate([B_, bB[None, :], zf], axis=0)

    Wout = Wout_ref[...]
    bout = bout_ref[...]
    xm = x_ref[...]
    am = acc_ref[0] + acc_ref[1]
    for j in range(8):
        x_j = xm[:, j * 16:(j + 1) * 16]
        a_j = am[:, j * 16:(j + 1) * 16]
        cntc = jnp.maximum(a_j[:, 13:14], 1.0)
        hs = (jnp.dot(x_j, A16, preferred_element_type=jnp.float32)
              + jnp.dot(a_j, B16, preferred_element_type=jnp.float32) / cntc)
        mu = jnp.mean(hs, axis=1, keepdims=True)
        var = jnp.mean((hs - mu) * (hs - mu), axis=1, keepdims=True)
        hn = (hs - mu) * lax.rsqrt(var + 1e-5)
        ha = jnp.where(hn > 0, hn, 0.01 * hn)
        o_ref[:, j * T_OUT:(j + 1) * T_OUT] = (
            jnp.dot(ha, Wout, preferred_element_type=jnp.float32) + bout)


def _tc_dense(xm, accm, W_in, b_in, W_orin, b_orin, W_self, W_neigh,
              b_sage, W_out, b_out):
    full = lambda shape: pl.BlockSpec(shape, lambda: (0,) * len(shape))
    return pl.pallas_call(
        _tc_body,
        grid=(),
        in_specs=[
            full((NR8, 128)),
            full((NC, NR8, 128)),
            full((T_IN, U)),
            full((U,)),
            full((U, U)),
            full((U,)),
            full((U, U)),
            full((U, U)),
            full((U,)),
            full((U, T_OUT)),
            full((T_OUT,)),
        ],
        out_specs=full((NR8, 8 * T_OUT)),
        out_shape=jax.ShapeDtypeStruct((NR8, 8 * T_OUT), jnp.float32),
    )(xm, accm, W_in, b_in, W_orin, b_orin, W_self, W_neigh, b_sage,
      W_out, b_out)


def kernel(x, edge_index, edge_weight, W_in, b_in, W_orin, b_orin, W_self,
           W_neigh, b_sage, W_out, b_out):
    x_pad = jnp.concatenate(
        [x, jnp.ones((N, 1), jnp.float32), jnp.zeros((N, 3), jnp.float32)],
        axis=1)
    x_pad = jnp.pad(x_pad, ((0, NPAD - N), (0, 0)))
    edge_r = edge_index.reshape(2, NW, RPT, CH)
    w_r = edge_weight.reshape(NW, RPT, CH)
    acc = _sc_aggregate(x_pad, edge_r, w_r)
    xm = x_pad.reshape(NR8, 128)
    accm = acc.reshape(NC, NR8, 128)
    out = _tc_dense(xm, accm, W_in, b_in, W_orin, b_orin, W_self, W_neigh,
                    b_sage, W_out, b_out)
    return out.reshape(NPAD, T_OUT)[:N]


# speedup vs baseline: 27.1308x; 1.0551x over previous
import functools

import jax
import jax.numpy as jnp
from jax import lax
from jax.experimental import pallas as pl
from jax.experimental.pallas import tpu as pltpu
from jax.experimental.pallas import tpu_sc as plsc

N = 10000
E = 320000
U = 128
T_IN = 12
T_OUT = 12

NC = 2
NS = 16
NW = NC * NS
CH = 80
EROWS = E // CH
RPT = EROWS // NW
NPAD = 10240
NPT = NPAD // NS
NBUF = 5

_GATHER_DNUMS = lax.GatherDimensionNumbers(
    offset_dims=(), collapsed_slice_dims=(0,), start_index_map=(0,))


def _sc_aggregate(x_pad, edge_r, w_r):
    mesh = plsc.VectorSubcoreMesh(core_axis_name="c", subcore_axis_name="s")

    @functools.partial(
        pl.kernel,
        mesh=mesh,
        compiler_params=pltpu.CompilerParams(use_tc_tiling_on_sc=False),
        out_type=jax.ShapeDtypeStruct((NC, NS, NPT, 16), jnp.float32),
        scratch_types=[
            pltpu.VMEM((RPT, CH), jnp.int32),
            pltpu.VMEM((RPT, CH), jnp.int32),
            pltpu.VMEM((RPT, CH), jnp.float32),
            [pltpu.VMEM((CH, 16), jnp.float32)] * NBUF,
            [pltpu.VMEM((CH, 16), jnp.float32)] * NBUF,
            pltpu.VMEM((NPT, 16), jnp.float32),
            pltpu.VMEM((16,), jnp.float32),
            pltpu.VMEM_SHARED((NPAD, 16), jnp.float32),
            [pltpu.SemaphoreType.DMA] * NBUF,
            [pltpu.SemaphoreType.DMA] * NBUF,
        ],
    )
    def agg(x_hbm, edge_hbm, w_hbm, m13_hbm, out_hbm,
            src_v, dst_v, w_v, gbufs, sbufs, zbuf, m13_v, acc_sh,
            gsems, ssems):
        c = lax.axis_index("c")
        s = lax.axis_index("s")
        zero16 = jnp.zeros((16,), jnp.float32)
        pltpu.sync_copy(m13_hbm, m13_v)

        def zrow(i, carry):
            zbuf[i] = zero16
            return carry
        lax.fori_loop(0, NPT, zrow, 0)
        pltpu.sync_copy(zbuf, acc_sh.at[pl.ds(s * NPT, NPT)])

        g = c * NS + s
        pltpu.sync_copy(edge_hbm.at[0, g], src_v)
        pltpu.sync_copy(edge_hbm.at[1, g], dst_v)
        pltpu.sync_copy(w_hbm.at[g], w_v)
        def zsc(i, carry):
            for b in range(NBUF):
                sbufs[b][i] = zero16
            return carry
        lax.fori_loop(0, CH, zsc, 0)
        plsc.subcore_barrier()

        def fire(chunk_ix, b):
            if isinstance(chunk_ix, int):
                pltpu.async_copy(x_hbm.at[src_v.at[chunk_ix]], gbufs[b],
                                 gsems[b])
            else:
                @pl.when(chunk_ix < RPT)
                def _():
                    pltpu.async_copy(x_hbm.at[src_v.at[chunk_ix]], gbufs[b],
                                     gsems[b])

        def gwait(b):
            pltpu.make_async_copy(x_hbm.at[src_v.at[0]], gbufs[b],
                                  gsems[b]).wait()

        def scat(b, chunk_ix):
            pltpu.async_copy(sbufs[b], acc_sh.at[dst_v.at[chunk_ix]],
                             ssems[b], add=True)

        def swait(b):
            pltpu.make_async_copy(sbufs[b], acc_sh.at[dst_v.at[0]],
                                  ssems[b]).wait()

        def process(b, chunk_ix):
            cm13 = m13_v[...]
            gbuf = gbufs[b]
            sbuf = sbufs[b]
            for k in range(CH // 16):
                w16 = w_v[chunk_ix, pl.ds(k * 16, 16)]
                for e in range(16):
                    wv = lax.gather(
                        w16, jnp.full((16, 1), e, jnp.int32),
                        _GATHER_DNUMS, (1,),
                        mode=lax.GatherScatterMode.PROMISE_IN_BOUNDS)
                    ee = k * 16 + e
                    sbuf[ee] = gbuf[ee] * wv + cm13

        for b in range(NBUF):
            scat(b, 0)
        for b in range(NBUF - 1):
            fire(b, b)

        def body(i, carry):
            c0 = NBUF * i
            fire(c0 + NBUF - 1, NBUF - 1)
            for b in range(NBUF):
                gwait(b)
                swait(b)
                process(b, c0 + b)
                scat(b, c0 + b)
                if b < NBUF - 1:
                    fire(c0 + NBUF + b, b)
            return carry
        lax.fori_loop(0, RPT // NBUF, body, 0)
        for b in range(NBUF):
            swait(b)

        plsc.subcore_barrier()
        pltpu.sync_copy(acc_sh.at[pl.ds(s * NPT, NPT)],
                        out_hbm.at[c, s])

    m13 = jnp.zeros((16,), jnp.float32).at[13].set(1.0)
    return agg(x_pad, edge_r, w_r, m13)


NR8 = NPAD // 8


def _tc_body(x_ref, acc_ref, Win_ref, bin_ref, Worin_ref, borin_ref,
             Wself_ref, Wneigh_ref, bsage_ref, Wout_ref, bout_ref, o_ref):
    Win = Win_ref[...]
    Worin = Worin_ref[...]
    W2 = jnp.dot(Win, Worin, preferred_element_type=jnp.float32)
    b2 = jnp.dot(bin_ref[...][None, :], Worin,
                 preferred_element_type=jnp.float32)[0] + borin_ref[...]
    A_ = jnp.dot(W2, Wself_ref[...], preferred_element_type=jnp.float32)
    B_ = jnp.dot(W2, Wneigh_ref[...], preferred_element_type=jnp.float32)
    bC = (jnp.dot(b2[None, :], Wself_ref[...],
                  preferred_element_type=jnp.float32)[0] + bsage_ref[...])
    bB = jnp.dot(b2[None, :], Wneigh_ref[...],
                 preferred_element_type=jnp.float32)[0]
    zf = jnp.zeros((3, U), jnp.float32)
    A16 = jnp.concatenate([A_, bC[None, :], zf], axis=0)
    B16 = jnp.concatenate([B_, bB[None, :], zf], axis=0)

    Wout = Wout_ref[...]
    bout = bout_ref[...]
    xm = x_ref[...]
    am = acc_ref[0] + acc_ref[1]
    for j in range(8):
        x_j = xm[:, j * 16:(j + 1) * 16]
        a_j = am[:, j * 16:(j + 1) * 16]
        cntc = jnp.maximum(a_j[:, 13:14], 1.0)
        hs = (jnp.dot(x_j, A16, preferred_element_type=jnp.float32)
              + jnp.dot(a_j, B16, preferred_element_type=jnp.float32) / cntc)
        mu = jnp.mean(hs, axis=1, keepdims=True)
        var = jnp.mean((hs - mu) * (hs - mu), axis=1, keepdims=True)
        hn = (hs - mu) * lax.rsqrt(var + 1e-5)
        ha = jnp.where(hn > 0, hn, 0.01 * hn)
        o_ref[:, j * T_OUT:(j + 1) * T_OUT] = (
            jnp.dot(ha, Wout, preferred_element_type=jnp.float32) + bout)


def _tc_dense(xm, accm, W_in, b_in, W_orin, b_orin, W_self, W_neigh,
              b_sage, W_out, b_out):
    full = lambda shape: pl.BlockSpec(shape, lambda: (0,) * len(shape))
    return pl.pallas_call(
        _tc_body,
        grid=(),
        in_specs=[
            full((NR8, 128)),
            full((NC, NR8, 128)),
            full((T_IN, U)),
            full((U,)),
            full((U, U)),
            full((U,)),
            full((U, U)),
            full((U, U)),
            full((U,)),
            full((U, T_OUT)),
            full((T_OUT,)),
        ],
        out_specs=full((NR8, 8 * T_OUT)),
        out_shape=jax.ShapeDtypeStruct((NR8, 8 * T_OUT), jnp.float32),
    )(xm, accm, W_in, b_in, W_orin, b_orin, W_self, W_neigh, b_sage,
      W_out, b_out)


def kernel(x, edge_index, edge_weight, W_in, b_in, W_orin, b_orin, W_self,
           W_neigh, b_sage, W_out, b_out):
    x_pad = jnp.concatenate(
        [x, jnp.ones((N, 1), jnp.float32), jnp.zeros((N, 3), jnp.float32)],
        axis=1)
    x_pad = jnp.pad(x_pad, ((0, NPAD - N), (0, 0)))
    edge_r = edge_index.reshape(2, NW, RPT, CH)
    w_r = edge_weight.reshape(NW, RPT, CH)
    acc = _sc_aggregate(x_pad, edge_r, w_r)
    xm = x_pad.reshape(NR8, 128)
    accm = acc.reshape(NC, NR8, 128)
    out = _tc_dense(xm, accm, W_in, b_in, W_orin, b_orin, W_self, W_neigh,
                    b_sage, W_out, b_out)
    return out.reshape(NPAD, T_OUT)[:N]
